# baseline (speedup 1.0000x reference)
import functools

import jax
import jax.numpy as jnp
from jax import lax
from jax.experimental import pallas as pl
from jax.experimental.pallas import tpu as pltpu
from jax.experimental.pallas import tpu_sc as plsc

EMBED = 32
LANES = 16
WIN = 256
RING = 32
ROWPAD = 128


def kernel(users, items, user_table, item_table):
    users = users.astype(jnp.int32)
    items = items.astype(jnp.int32)
    ut = user_table.T
    it = item_table.T
    batch = users.shape[0]
    n = user_table.shape[0]

    nfull = n // WIN
    tail = n - nfull * WIN

    info = plsc.get_sparse_core_info()
    nc, ns = info.num_cores, info.num_subcores
    nw = nc * ns
    bpw = batch // nw
    wper = nfull // nw
    wextra = nfull - wper * nw
    brows = batch + 8

    mesh = plsc.VectorSubcoreMesh(core_axis_name="c", subcore_axis_name="s")
    lanes = lambda: lax.iota(jnp.int32, LANES)

    @functools.partial(
        pl.kernel,
        mesh=mesh,
        compiler_params=pltpu.CompilerParams(needs_layout_passes=False),
        out_type=(jax.ShapeDtypeStruct((brows, ROWPAD), jnp.float32),
                  jax.ShapeDtypeStruct((brows, ROWPAD), jnp.float32)),
        scratch_types=[
            pltpu.VMEM((batch,), jnp.int32),
            pltpu.VMEM((batch,), jnp.int32),
            pltpu.VMEM((batch,), jnp.int32),
            pltpu.VMEM((batch,), jnp.int32),
            pltpu.VMEM((2, EMBED, WIN), jnp.float32),
            pltpu.VMEM((2, EMBED, WIN), jnp.float32),
            pltpu.VMEM((2, RING, ROWPAD), jnp.float32),
            pltpu.VMEM((2, RING, ROWPAD), jnp.float32),
            pltpu.VMEM((2, RING), jnp.int32),
            pltpu.VMEM((2, RING), jnp.int32),
            pltpu.VMEM((EMBED, tail or 1), jnp.float32),
            pltpu.VMEM((EMBED, tail or 1), jnp.float32),
            pltpu.SemaphoreType.DMA((2,)),
            pltpu.SemaphoreType.DMA,
            pltpu.SemaphoreType.DMA,
        ],
    )
    def scan(users_hbm, items_hbm, ut_hbm, it_hbm, urows_hbm, irows_hbm,
             uidxv, iidxv, uhits, ihits, uwin, iwin, uring, iring,
             uridx, iridx, utail, itail, wsem, fsemu, fsemi):
        wid = lax.axis_index("s") * nc + lax.axis_index("c")
        lo = wid * wper + jnp.minimum(wid, wextra)
        cnt_w = wper + jnp.where(wid < wextra, 1, 0)
        clo = lo * WIN
        chi = (lo + cnt_w) * WIN
        is_last = wid == nw - 1
        chi_eff = jnp.where(is_last, n, chi)

        pltpu.sync_copy(users_hbm, uidxv)
        pltpu.sync_copy(items_hbm, iidxv)

        def hitscan(ref, hits):
            def body(vi, cnt):
                off = pl.multiple_of(vi * LANES, LANES)
                r = ref[pl.ds(off, LANES)]
                mine = jnp.logical_and(r >= clo, r < chi_eff)
                m32 = jnp.where(mine, 1, 0)
                pos = cnt + plsc.cumsum(m32) - 1
                plsc.store_scatter(hits, [pos], off + lanes(), mask=mine)
                return cnt + plsc.all_reduce_population_count(mine)
            cnt = lax.fori_loop(0, batch // LANES, body,
                                jnp.zeros((LANES,), jnp.int32))
            return lax.reduce_max(cnt, (0,))

        ucnt = hitscan(uidxv, uhits)
        icnt = hitscan(iidxv, ihits)

        for p in range(2):
            for k in range(RING // LANES):
                uridx[p, pl.ds(k * LANES, LANES)] = jnp.full(
                    (LANES,), batch, jnp.int32)
                iridx[p, pl.ds(k * LANES, LANES)] = jnp.full(
                    (LANES,), batch, jnp.int32)

        def fire(k):
            c0 = pl.multiple_of((lo + k) * WIN, 128)
            buf = lax.rem(k, 2)
            pltpu.async_copy(ut_hbm.at[:, pl.ds(c0, WIN)], uwin.at[buf],
                             wsem.at[buf])
            pltpu.async_copy(it_hbm.at[:, pl.ds(c0, WIN)], iwin.at[buf],
                             wsem.at[buf])

        fire(0)

        def extract_table(win_ref, hits, hcnt, idxv, ring, ridx, rows_hbm,
                          fsem, c0, wlen, par, rpos, pend):
            gmax = lax.div(hcnt + (LANES - 1), LANES)

            def gbody(g, carry):
                par, rpos, pend = carry
                goff = g * LANES
                hb = plsc.load_gather(hits, [goff + lanes()])
                valid = (goff + lanes()) < hcnt
                r = plsc.load_gather(idxv, [jnp.where(valid, hb, 0)])
                inwin = jnp.logical_and(
                    valid, jnp.logical_and(r >= c0, r < c0 + wlen))
                nhit = plsc.all_reduce_population_count(inwin)
                any_hit = lax.reduce_max(nhit, (0,)) > 0

                def do_extract(par, rpos, pend):
                    need_flush = lax.reduce_max(rpos, (0,)) > RING - LANES

                    def flush(par, rpos, pend):
                        @pl.when(pend > 0)
                        def _():
                            pltpu.make_async_copy(
                                ring.at[lax.rem(par + 1, 2)],
                                rows_hbm.at[pl.ds(0, RING)], fsem).wait()
                        pltpu.async_copy(
                            ring.at[par], rows_hbm.at[ridx.at[par]], fsem)
                        newpar = lax.rem(par + 1, 2)
                        for k in range(RING // LANES):
                            ridx[newpar, pl.ds(k * LANES, LANES)] = jnp.full(
                                (LANES,), batch, jnp.int32)
                        return newpar, jnp.zeros((LANES,), jnp.int32), pend + 1

                    par, rpos, pend = lax.cond(
                        need_flush, flush,
                        lambda a, b, c: (a, b, c), par, rpos, pend)

                    col = r - c0
                    slot = rpos + plsc.cumsum(jnp.where(inwin, 1, 0)) - 1
                    pvec = jnp.full((LANES,), 0, jnp.int32) + par
                    for d in range(EMBED):
                        dvec = jnp.full((LANES,), d, jnp.int32)
                        vals = plsc.load_gather(
                            win_ref, [dvec, jnp.where(inwin, col, 0)])
                        plsc.store_scatter(ring, [pvec, slot, dvec], vals,
                                           mask=inwin)
                    plsc.store_scatter(ridx, [pvec, slot], hb, mask=inwin)
                    return par, rpos + nhit, pend

                return lax.cond(any_hit, do_extract,
                                lambda a, b, c: (a, b, c), par, rpos, pend)

            return lax.fori_loop(0, gmax, gbody, (par, rpos, pend))

        def wloop(k, carry):
            up, ur, upend, ip, ir, ipend = carry
            cur = lax.rem(k, 2)

            @pl.when(k + 1 < cnt_w)
            def _():
                fire(k + 1)

            pltpu.make_async_copy(ut_hbm.at[:, pl.ds(0, WIN)], uwin.at[cur],
                                  wsem.at[cur]).wait()
            pltpu.make_async_copy(it_hbm.at[:, pl.ds(0, WIN)], iwin.at[cur],
                                  wsem.at[cur]).wait()

            c0 = (lo + k) * WIN
            up, ur, upend = extract_table(
                uwin.at[cur], uhits, ucnt, uidxv, uring, uridx, urows_hbm,
                fsemu, c0, WIN, up, ur, upend)
            ip, ir, ipend = extract_table(
                iwin.at[cur], ihits, icnt, iidxv, iring, iridx, irows_hbm,
                fsemi, c0, WIN, ip, ir, ipend)
            return up, ur, upend, ip, ir, ipend

        zero = jnp.zeros((LANES,), jnp.int32)
        carry = lax.fori_loop(
            0, cnt_w, wloop, (0, zero, 0, 0, zero, 0))
        up, ur, upend, ip, ir, ipend = carry

        if tail:
            def do_tail(up, ur, upend, ip, ir, ipend):
                c0 = nfull * WIN
                pltpu.sync_copy(ut_hbm.at[:, pl.ds(c0, tail)], utail)
                pltpu.sync_copy(it_hbm.at[:, pl.ds(c0, tail)], itail)
                up, ur, upend = extract_table(
                    utail, uhits, ucnt, uidxv, uring, uridx, urows_hbm,
                    fsemu, c0, tail, up, ur, upend)
                ip, ir, ipend = extract_table(
                    itail, ihits, icnt, iidxv, iring, iridx, irows_hbm,
                    fsemi, c0, tail, ip, ir, ipend)
                return up, ur, upend, ip, ir, ipend

            up, ur, upend, ip, ir, ipend = lax.cond(
                is_last, do_tail,
                lambda a, b, c, d, e, f: (a, b, c, d, e, f),
                up, ur, upend, ip, ir, ipend)

        def finalize(ring, ridx, rows_hbm, fsem, par, rpos, pend):
            has_rows = lax.reduce_max(rpos, (0,)) > 0

            @pl.when(jnp.logical_and(has_rows, pend > 0))
            def _():
                pltpu.make_async_copy(ring.at[lax.rem(par + 1, 2)],
                                      rows_hbm.at[pl.ds(0, RING)],
                                      fsem).wait()

            @pl.when(has_rows)
            def _():
                pltpu.async_copy(ring.at[par], rows_hbm.at[ridx.at[par]],
                                 fsem)

            pend = jnp.where(has_rows, 1, jnp.minimum(pend, 1))

            @pl.when(pend > 0)
            def _():
                pltpu.make_async_copy(ring.at[0],
                                      rows_hbm.at[pl.ds(0, RING)],
                                      fsem).wait()

        finalize(uring, uridx, urows_hbm, fsemu, up, ur, upend)
        finalize(iring, iridx, irows_hbm, fsemi, ip, ir, ipend)

    CH = 128

    @functools.partial(
        pl.kernel,
        mesh=mesh,
        compiler_params=pltpu.CompilerParams(needs_layout_passes=False),
        out_type=jax.ShapeDtypeStruct((batch,), jnp.float32),
        scratch_types=[
            pltpu.VMEM((CH, ROWPAD), jnp.float32),
            pltpu.VMEM((CH, ROWPAD), jnp.float32),
            pltpu.VMEM((bpw,), jnp.float32),
        ],
    )
    def dots(urows_hbm, irows_hbm, out_hbm, uch, ich, outv):
        wid = lax.axis_index("s") * nc + lax.axis_index("c")
        base = wid * bpw

        def chunk(j, carry):
            row0 = pl.multiple_of(base + j * CH, 8)
            pltpu.sync_copy(urows_hbm.at[pl.ds(row0, CH)], uch)
            pltpu.sync_copy(irows_hbm.at[pl.ds(row0, CH)], ich)

            def group(g, c2):
                rvec = g * LANES + lanes()
                acc = jnp.zeros((LANES,), jnp.float32)
                for d in range(EMBED):
                    dvec = jnp.full((LANES,), d, jnp.int32)
                    uv = plsc.load_gather(uch, [rvec, dvec])
                    iv = plsc.load_gather(ich, [rvec, dvec])
                    acc = acc + uv * iv
                off = pl.multiple_of(j * CH + g * LANES, LANES)
                outv[pl.ds(off, LANES)] = acc
                return c2

            lax.fori_loop(0, CH // LANES, group, 0)
            return carry

        lax.fori_loop(0, bpw // CH, chunk, 0)
        pltpu.sync_copy(outv, out_hbm.at[pl.ds(base, bpw)])

    urows, irows = scan(users, items, ut, it)
    return dots(urows, irows)
